# baseline (device time: 169235 ns/iter reference)
import jax
import jax.numpy as jnp
from jax import lax
from jax.experimental import pallas as pl
from jax.experimental.pallas import tpu as pltpu

N_DEV = 4


def kernel(A, B):
    m, _ = A.shape
    _, n = B.shape
    hm, hn = m // 2, n // 2
    qm = m // 4

    def body(a_ref, b_ref, z_ref, out_ref, q0, q1, k0, k1, c1h0, c1h1,
             c2h0, c2h1, send_sems, recv_sems, copy_sems):
        del z_ref
        my = lax.axis_index("i")
        h1 = lax.bitwise_and(lax.bitwise_xor(my, lax.shift_right_logical(my, 1)), 1)
        h1p = lax.bitwise_and(lax.shift_right_logical(my, 1), 1)
        bit0 = lax.bitwise_and(my, 1)
        c0 = 2 * h1 + h1p
        p1 = lax.bitwise_xor(my, 1)
        p2 = 3 - my

        barrier_sem = pltpu.get_barrier_semaphore()
        for nbr in (p1, p2):
            pl.semaphore_signal(
                barrier_sem, inc=1,
                device_id=(nbr,), device_id_type=pl.DeviceIdType.MESH,
            )
        pl.semaphore_wait(barrier_sem, 2)

        def exchange(src, dst, sem_idx, partner):
            return pltpu.make_async_remote_copy(
                src_ref=src, dst_ref=dst,
                send_sem=send_sems.at[sem_idx], recv_sem=recv_sems.at[sem_idx],
                device_id=(partner,), device_id_type=pl.DeviceIdType.MESH,
            )

        send_r0 = (1 - h1) * hm
        send_r1 = (1 - h1p) * hm
        off0a = (1 - h1p) * qm
        off0b = h1p * qm
        off1a = bit0 * qm
        off1b = (1 - bit0) * qm

        def half_dot(dst_ref, dst_off, a_row, b_col):
            dst_ref[pl.ds(dst_off, qm), :] = jnp.dot(
                a_ref[pl.ds(a_row, qm), :],
                b_ref[:, pl.ds(b_col, hn)],
                preferred_element_type=jnp.float32,
            )

        half_dot(q0, off0a, send_r0 + off0a, 0)
        rd1a = exchange(q0.at[pl.ds(off0a, qm), :], c1h0.at[0], 0, p1)
        rd1a.start()
        half_dot(q1, off1a, send_r1 + off1a, hn)
        rd2a = exchange(q1.at[pl.ds(off1a, qm), :], c1h1.at[0], 1, p2)
        rd2a.start()
        half_dot(q0, off0b, send_r0 + off0b, 0)
        rd1b = exchange(q0.at[pl.ds(off0b, qm), :], c1h0.at[1], 2, p1)
        rd1b.start()
        half_dot(q1, off1b, send_r1 + off1b, hn)
        rd2b = exchange(q1.at[pl.ds(off1b, qm), :], c1h1.at[1], 3, p2)
        rd2b.start()

        keep_r0 = h1 * hm
        keep_r1 = h1p * hm
        k0[...] = jnp.dot(
            a_ref[pl.ds(keep_r0, hm), :], b_ref[:, pl.ds(0, hn)],
            preferred_element_type=jnp.float32,
        )
        k1[...] = jnp.dot(
            a_ref[pl.ds(keep_r1, hm), :], b_ref[:, pl.ds(hn, hn)],
            preferred_element_type=jnp.float32,
        )

        f0 = (1 - h1p) * qm
        o0 = h1p * qm
        f1 = (1 - bit0) * qm
        o1 = bit0 * qm

        rd1a.wait()
        k0[pl.ds(f0, qm), :] += c1h0[0]
        rd20 = exchange(k0.at[pl.ds(f0, qm), :], c2h0, 4, p2)
        rd20.start()
        rd2a.wait()
        k1[pl.ds(f1, qm), :] += c1h1[0]
        rd21 = exchange(k1.at[pl.ds(f1, qm), :], c2h1, 5, p1)
        rd21.start()

        rd1b.wait()
        k0[pl.ds(o0, qm), :] += c1h0[1]
        rd2b.wait()
        k1[pl.ds(o1, qm), :] += c1h1[1]

        rd20.wait()
        k0[pl.ds(o0, qm), :] += c2h0[...]
        rd30 = exchange(k0.at[pl.ds(o0, qm), :], k0.at[pl.ds(o0, qm), :], 6, p2)
        rd30.start()
        rd21.wait()
        k1[pl.ds(o1, qm), :] += c2h1[...]
        rd31 = exchange(k1.at[pl.ds(o1, qm), :], k1.at[pl.ds(o1, qm), :], 7, p1)
        rd31.start()

        rd4a0 = exchange(k0.at[pl.ds(o0, qm), :],
                         out_ref.at[pl.ds(c0 * qm, qm), pl.ds(0, hn)], 8, p1)
        rd4a0.start()
        rd4a1 = exchange(k1.at[pl.ds(o1, qm), :],
                         out_ref.at[pl.ds(my * qm, qm), pl.ds(hn, hn)], 9, p2)
        rd4a1.start()

        s2_0 = (2 * h1 + 1 - h1p) * qm
        s2_1 = (2 * h1p + 1 - bit0) * qm
        rd30.wait()
        rd4b0 = exchange(k0.at[pl.ds(f0, qm), :],
                         out_ref.at[pl.ds(s2_0, qm), pl.ds(0, hn)], 10, p1)
        rd4b0.start()
        cp0 = pltpu.make_async_copy(
            k0, out_ref.at[pl.ds(keep_r0, hm), pl.ds(0, hn)], copy_sems.at[0]
        )
        cp0.start()
        rd31.wait()
        rd4b1 = exchange(k1.at[pl.ds(f1, qm), :],
                         out_ref.at[pl.ds(s2_1, qm), pl.ds(hn, hn)], 11, p2)
        rd4b1.start()
        cp1 = pltpu.make_async_copy(
            k1, out_ref.at[pl.ds(keep_r1, hm), pl.ds(hn, hn)], copy_sems.at[1]
        )
        cp1.start()

        rd4a0.wait()
        rd4a1.wait()
        rd4b0.wait()
        rd4b1.wait()
        cp0.wait()
        cp1.wait()

    zero = jnp.zeros((m, n), jnp.float32)
    return pl.pallas_call(
        body,
        out_shape=jax.ShapeDtypeStruct((m, n), jnp.float32),
        in_specs=[
            pl.BlockSpec(memory_space=pltpu.VMEM),
            pl.BlockSpec(memory_space=pltpu.VMEM),
            pl.BlockSpec(memory_space=pl.ANY),
        ],
        out_specs=pl.BlockSpec(memory_space=pl.ANY),
        input_output_aliases={2: 0},
        scratch_shapes=[
            pltpu.VMEM((hm, hn), jnp.float32),
            pltpu.VMEM((hm, hn), jnp.float32),
            pltpu.VMEM((hm, hn), jnp.float32),
            pltpu.VMEM((hm, hn), jnp.float32),
            pltpu.VMEM((2, qm, hn), jnp.float32),
            pltpu.VMEM((2, qm, hn), jnp.float32),
            pltpu.VMEM((qm, hn), jnp.float32),
            pltpu.VMEM((qm, hn), jnp.float32),
            pltpu.SemaphoreType.DMA((12,)),
            pltpu.SemaphoreType.DMA((12,)),
            pltpu.SemaphoreType.DMA((2,)),
        ],
        compiler_params=pltpu.CompilerParams(
            collective_id=0,
            vmem_limit_bytes=60 * 1024 * 1024,
        ),
    )(A, B, zero)


# device time: 161695 ns/iter; 1.0466x vs baseline; 1.0466x over previous
import jax
import jax.numpy as jnp
from jax import lax
from jax.experimental import pallas as pl
from jax.experimental.pallas import tpu as pltpu

N_DEV = 4


def kernel(A, B):
    m, _ = A.shape
    _, n = B.shape
    hm, hn = m // 2, n // 2
    qm = m // 4

    def body(a_ref, b_ref, out_ref, q0, q1, k0, k1, c1h0, c1h1,
             c2h0, c2h1, send_sems, recv_sems, copy_sems):
        my = lax.axis_index("i")
        h1 = lax.bitwise_and(lax.bitwise_xor(my, lax.shift_right_logical(my, 1)), 1)
        h1p = lax.bitwise_and(lax.shift_right_logical(my, 1), 1)
        bit0 = lax.bitwise_and(my, 1)
        c0 = 2 * h1 + h1p
        p1 = lax.bitwise_xor(my, 1)
        p2 = 3 - my

        barrier_sem = pltpu.get_barrier_semaphore()
        for nbr in (p1, p2):
            pl.semaphore_signal(
                barrier_sem, inc=1,
                device_id=(nbr,), device_id_type=pl.DeviceIdType.MESH,
            )
        pl.semaphore_wait(barrier_sem, 2)

        def exchange(src, dst, sem_idx, partner):
            return pltpu.make_async_remote_copy(
                src_ref=src, dst_ref=dst,
                send_sem=send_sems.at[sem_idx], recv_sem=recv_sems.at[sem_idx],
                device_id=(partner,), device_id_type=pl.DeviceIdType.MESH,
            )

        send_r0 = (1 - h1) * hm
        send_r1 = (1 - h1p) * hm
        off0a = (1 - h1p) * qm
        off0b = h1p * qm
        off1a = bit0 * qm
        off1b = (1 - bit0) * qm

        def piece_dot(dst_ref, dst_off, a_row, nrows, b_col):
            dst_ref[pl.ds(dst_off, nrows), :] = jnp.dot(
                a_ref[pl.ds(a_row, nrows), :],
                b_ref[:, pl.ds(b_col, hn)],
                preferred_element_type=jnp.float32,
            )

        em = qm // 2
        piece_dot(q0, off0a, send_r0 + off0a, em, 0)
        rd1a1 = exchange(q0.at[pl.ds(off0a, em), :],
                         c1h0.at[0, pl.ds(0, em), :], 12, p1)
        rd1a1.start()
        piece_dot(q1, off1a, send_r1 + off1a, em, hn)
        rd2a1 = exchange(q1.at[pl.ds(off1a, em), :],
                         c1h1.at[0, pl.ds(0, em), :], 13, p2)
        rd2a1.start()
        piece_dot(q0, off0a + em, send_r0 + off0a + em, em, 0)
        rd1a = exchange(q0.at[pl.ds(off0a + em, em), :],
                        c1h0.at[0, pl.ds(em, em), :], 0, p1)
        rd1a.start()
        piece_dot(q1, off1a + em, send_r1 + off1a + em, em, hn)
        rd2a = exchange(q1.at[pl.ds(off1a + em, em), :],
                        c1h1.at[0, pl.ds(em, em), :], 1, p2)
        rd2a.start()
        piece_dot(q0, off0b, send_r0 + off0b, qm, 0)
        rd1b = exchange(q0.at[pl.ds(off0b, qm), :], c1h0.at[1], 2, p1)
        rd1b.start()
        piece_dot(q1, off1b, send_r1 + off1b, qm, hn)
        rd2b = exchange(q1.at[pl.ds(off1b, qm), :], c1h1.at[1], 3, p2)
        rd2b.start()

        keep_r0 = h1 * hm
        keep_r1 = h1p * hm
        k0[...] = jnp.dot(
            a_ref[pl.ds(keep_r0, hm), :], b_ref[:, pl.ds(0, hn)],
            preferred_element_type=jnp.float32,
        )
        k1[...] = jnp.dot(
            a_ref[pl.ds(keep_r1, hm), :], b_ref[:, pl.ds(hn, hn)],
            preferred_element_type=jnp.float32,
        )

        f0 = (1 - h1p) * qm
        o0 = h1p * qm
        f1 = (1 - bit0) * qm
        o1 = bit0 * qm

        rd1a1.wait()
        rd1a.wait()
        k0[pl.ds(f0, qm), :] += c1h0[0]
        rd20 = exchange(k0.at[pl.ds(f0, qm), :], c2h0, 4, p2)
        rd20.start()
        rd2a1.wait()
        rd2a.wait()
        k1[pl.ds(f1, qm), :] += c1h1[0]
        rd21 = exchange(k1.at[pl.ds(f1, qm), :], c2h1, 5, p1)
        rd21.start()

        rd1b.wait()
        k0[pl.ds(o0, qm), :] += c1h0[1]
        rd2b.wait()
        k1[pl.ds(o1, qm), :] += c1h1[1]

        rd20.wait()
        k0[pl.ds(o0, qm), :] += c2h0[...]
        rd30 = exchange(k0.at[pl.ds(o0, qm), :], k0.at[pl.ds(o0, qm), :], 6, p2)
        rd30.start()
        rd21.wait()
        k1[pl.ds(o1, qm), :] += c2h1[...]
        rd31 = exchange(k1.at[pl.ds(o1, qm), :], k1.at[pl.ds(o1, qm), :], 7, p1)
        rd31.start()

        rd4a0 = exchange(k0.at[pl.ds(o0, qm), :],
                         out_ref.at[pl.ds(c0 * qm, qm), pl.ds(0, hn)], 8, p1)
        rd4a0.start()
        rd4a1 = exchange(k1.at[pl.ds(o1, qm), :],
                         out_ref.at[pl.ds(my * qm, qm), pl.ds(hn, hn)], 9, p2)
        rd4a1.start()

        s2_0 = (2 * h1 + 1 - h1p) * qm
        s2_1 = (2 * h1p + 1 - bit0) * qm
        rd30.wait()
        rd4b0 = exchange(k0.at[pl.ds(f0, qm), :],
                         out_ref.at[pl.ds(s2_0, qm), pl.ds(0, hn)], 10, p1)
        rd4b0.start()
        cp0 = pltpu.make_async_copy(
            k0, out_ref.at[pl.ds(keep_r0, hm), pl.ds(0, hn)], copy_sems.at[0]
        )
        cp0.start()
        rd31.wait()
        rd4b1 = exchange(k1.at[pl.ds(f1, qm), :],
                         out_ref.at[pl.ds(s2_1, qm), pl.ds(hn, hn)], 11, p2)
        rd4b1.start()
        cp1 = pltpu.make_async_copy(
            k1, out_ref.at[pl.ds(keep_r1, hm), pl.ds(hn, hn)], copy_sems.at[1]
        )
        cp1.start()

        rd4a0.wait()
        rd4a1.wait()
        rd4b0.wait()
        rd4b1.wait()
        cp0.wait()
        cp1.wait()

    return pl.pallas_call(
        body,
        out_shape=jax.ShapeDtypeStruct((m, n), jnp.float32),
        in_specs=[
            pl.BlockSpec(memory_space=pltpu.VMEM),
            pl.BlockSpec(memory_space=pltpu.VMEM),
        ],
        out_specs=pl.BlockSpec(memory_space=pl.ANY),
        scratch_shapes=[
            pltpu.VMEM((hm, hn), jnp.float32),
            pltpu.VMEM((hm, hn), jnp.float32),
            pltpu.VMEM((hm, hn), jnp.float32),
            pltpu.VMEM((hm, hn), jnp.float32),
            pltpu.VMEM((2, qm, hn), jnp.float32),
            pltpu.VMEM((2, qm, hn), jnp.float32),
            pltpu.VMEM((qm, hn), jnp.float32),
            pltpu.VMEM((qm, hn), jnp.float32),
            pltpu.SemaphoreType.DMA((14,)),
            pltpu.SemaphoreType.DMA((14,)),
            pltpu.SemaphoreType.DMA((2,)),
        ],
        compiler_params=pltpu.CompilerParams(
            collective_id=0,
            vmem_limit_bytes=60 * 1024 * 1024,
        ),
    )(A, B)


# device time: 161674 ns/iter; 1.0468x vs baseline; 1.0001x over previous
import jax
import jax.numpy as jnp
from jax import lax
from jax.experimental import pallas as pl
from jax.experimental.pallas import tpu as pltpu

N_DEV = 4


def kernel(A, B):
    m, _ = A.shape
    _, n = B.shape
    hm, hn = m // 2, n // 2
    qm = m // 4

    def body(a_ref, b_ref, out_ref, q0, q1, k0, k1, c1h0, c1h1,
             c2h0, c2h1, send_sems, recv_sems, copy_sems):
        my = lax.axis_index("i")
        h1 = lax.bitwise_and(lax.bitwise_xor(my, lax.shift_right_logical(my, 1)), 1)
        h1p = lax.bitwise_and(lax.shift_right_logical(my, 1), 1)
        bit0 = lax.bitwise_and(my, 1)
        c0 = 2 * h1 + h1p
        p1 = lax.bitwise_xor(my, 1)
        p2 = 3 - my

        barrier_sem = pltpu.get_barrier_semaphore()
        for nbr in (p1, p2):
            pl.semaphore_signal(
                barrier_sem, inc=1,
                device_id=(nbr,), device_id_type=pl.DeviceIdType.MESH,
            )
        pl.semaphore_wait(barrier_sem, 2)

        def exchange(src, dst, sem_idx, partner):
            return pltpu.make_async_remote_copy(
                src_ref=src, dst_ref=dst,
                send_sem=send_sems.at[sem_idx], recv_sem=recv_sems.at[sem_idx],
                device_id=(partner,), device_id_type=pl.DeviceIdType.MESH,
            )

        send_r0 = (1 - h1) * hm
        send_r1 = (1 - h1p) * hm
        off0a = (1 - h1p) * qm
        off0b = h1p * qm
        off1a = bit0 * qm
        off1b = (1 - bit0) * qm

        def piece_dot(dst_ref, dst_off, a_row, nrows, b_col):
            dst_ref[pl.ds(dst_off, nrows), :] = jnp.dot(
                a_ref[pl.ds(a_row, nrows), :],
                b_ref[:, pl.ds(b_col, hn)],
                preferred_element_type=jnp.float32,
            )

        em = qm // 2
        piece_dot(q0, off0a, send_r0 + off0a, em, 0)
        rd1a1 = exchange(q0.at[pl.ds(off0a, em), :],
                         c1h0.at[0, pl.ds(0, em), :], 12, p1)
        rd1a1.start()
        piece_dot(q1, off1a, send_r1 + off1a, em, hn)
        rd2a1 = exchange(q1.at[pl.ds(off1a, em), :],
                         c1h1.at[0, pl.ds(0, em), :], 13, p2)
        rd2a1.start()
        piece_dot(q0, off0a + em, send_r0 + off0a + em, em, 0)
        rd1a = exchange(q0.at[pl.ds(off0a + em, em), :],
                        c1h0.at[0, pl.ds(em, em), :], 0, p1)
        rd1a.start()
        piece_dot(q1, off1a + em, send_r1 + off1a + em, em, hn)
        rd2a = exchange(q1.at[pl.ds(off1a + em, em), :],
                        c1h1.at[0, pl.ds(em, em), :], 1, p2)
        rd2a.start()
        piece_dot(q0, off0b, send_r0 + off0b, qm, 0)
        rd1b = exchange(q0.at[pl.ds(off0b, qm), :], c1h0.at[1], 2, p1)
        rd1b.start()
        piece_dot(q1, off1b, send_r1 + off1b, qm, hn)
        rd2b = exchange(q1.at[pl.ds(off1b, qm), :], c1h1.at[1], 3, p2)
        rd2b.start()

        keep_r0 = h1 * hm
        keep_r1 = h1p * hm
        k0[...] = jnp.dot(
            a_ref[pl.ds(keep_r0, hm), :], b_ref[:, pl.ds(0, hn)],
            preferred_element_type=jnp.float32,
        )
        k1[...] = jnp.dot(
            a_ref[pl.ds(keep_r1, hm), :], b_ref[:, pl.ds(hn, hn)],
            preferred_element_type=jnp.float32,
        )

        f0 = (1 - h1p) * qm
        o0 = h1p * qm
        f1 = (1 - bit0) * qm
        o1 = bit0 * qm

        rd1a1.wait()
        rd1a.wait()
        k0[pl.ds(f0, qm), :] += c1h0[0]
        rd20a = exchange(k0.at[pl.ds(f0, em), :],
                         c2h0.at[pl.ds(0, em), :], 4, p2)
        rd20a.start()
        rd20b = exchange(k0.at[pl.ds(f0 + em, em), :],
                         c2h0.at[pl.ds(em, em), :], 14, p2)
        rd20b.start()
        rd2a1.wait()
        rd2a.wait()
        k1[pl.ds(f1, qm), :] += c1h1[0]
        rd21a = exchange(k1.at[pl.ds(f1, em), :],
                         c2h1.at[pl.ds(0, em), :], 5, p1)
        rd21a.start()
        rd21b = exchange(k1.at[pl.ds(f1 + em, em), :],
                         c2h1.at[pl.ds(em, em), :], 15, p1)
        rd21b.start()

        rd1b.wait()
        k0[pl.ds(o0, qm), :] += c1h0[1]
        rd2b.wait()
        k1[pl.ds(o1, qm), :] += c1h1[1]

        rd20a.wait()
        k0[pl.ds(o0, em), :] += c2h0[pl.ds(0, em), :]
        rd30a = exchange(k0.at[pl.ds(o0, em), :], k0.at[pl.ds(o0, em), :], 6, p2)
        rd30a.start()
        rd20b.wait()
        k0[pl.ds(o0 + em, em), :] += c2h0[pl.ds(em, em), :]
        rd30b = exchange(k0.at[pl.ds(o0 + em, em), :],
                         k0.at[pl.ds(o0 + em, em), :], 16, p2)
        rd30b.start()
        rd21a.wait()
        k1[pl.ds(o1, em), :] += c2h1[pl.ds(0, em), :]
        rd31a = exchange(k1.at[pl.ds(o1, em), :], k1.at[pl.ds(o1, em), :], 7, p1)
        rd31a.start()
        rd21b.wait()
        k1[pl.ds(o1 + em, em), :] += c2h1[pl.ds(em, em), :]
        rd31b = exchange(k1.at[pl.ds(o1 + em, em), :],
                         k1.at[pl.ds(o1 + em, em), :], 17, p1)
        rd31b.start()

        rd4a0 = exchange(k0.at[pl.ds(o0, qm), :],
                         out_ref.at[pl.ds(c0 * qm, qm), pl.ds(0, hn)], 8, p1)
        rd4a0.start()
        rd4a1 = exchange(k1.at[pl.ds(o1, qm), :],
                         out_ref.at[pl.ds(my * qm, qm), pl.ds(hn, hn)], 9, p2)
        rd4a1.start()

        s2_0 = (2 * h1 + 1 - h1p) * qm
        s2_1 = (2 * h1p + 1 - bit0) * qm
        rd30a.wait()
        rd30b.wait()
        rd4b0 = exchange(k0.at[pl.ds(f0, qm), :],
                         out_ref.at[pl.ds(s2_0, qm), pl.ds(0, hn)], 10, p1)
        rd4b0.start()
        cp0 = pltpu.make_async_copy(
            k0, out_ref.at[pl.ds(keep_r0, hm), pl.ds(0, hn)], copy_sems.at[0]
        )
        cp0.start()
        rd31a.wait()
        rd31b.wait()
        rd4b1 = exchange(k1.at[pl.ds(f1, qm), :],
                         out_ref.at[pl.ds(s2_1, qm), pl.ds(hn, hn)], 11, p2)
        rd4b1.start()
        cp1 = pltpu.make_async_copy(
            k1, out_ref.at[pl.ds(keep_r1, hm), pl.ds(hn, hn)], copy_sems.at[1]
        )
        cp1.start()

        rd4a0.wait()
        rd4a1.wait()
        rd4b0.wait()
        rd4b1.wait()
        cp0.wait()
        cp1.wait()

    return pl.pallas_call(
        body,
        out_shape=jax.ShapeDtypeStruct((m, n), jnp.float32),
        in_specs=[
            pl.BlockSpec(memory_space=pltpu.VMEM),
            pl.BlockSpec(memory_space=pltpu.VMEM),
        ],
        out_specs=pl.BlockSpec(memory_space=pl.ANY),
        scratch_shapes=[
            pltpu.VMEM((hm, hn), jnp.float32),
            pltpu.VMEM((hm, hn), jnp.float32),
            pltpu.VMEM((hm, hn), jnp.float32),
            pltpu.VMEM((hm, hn), jnp.float32),
            pltpu.VMEM((2, qm, hn), jnp.float32),
            pltpu.VMEM((2, qm, hn), jnp.float32),
            pltpu.VMEM((qm, hn), jnp.float32),
            pltpu.VMEM((qm, hn), jnp.float32),
            pltpu.SemaphoreType.DMA((18,)),
            pltpu.SemaphoreType.DMA((18,)),
            pltpu.SemaphoreType.DMA((2,)),
        ],
        compiler_params=pltpu.CompilerParams(
            collective_id=0,
            vmem_limit_bytes=60 * 1024 * 1024,
        ),
    )(A, B)
